# baseline (speedup 1.0000x reference)
import jax
import jax.numpy as jnp
from jax import lax
from jax.experimental import pallas as pl
from jax.experimental.pallas import tpu as pltpu
from jax.experimental.pallas import tpu_sc as plsc

_N = 10000
_E = 160000
_C = 128
_K = 27

_NC = 2
_NS = 16
_NW = _NC * _NS
_G = 128
_GR = 125
_ROWS = _E // _GR
_CH = _ROWS // _NW
_NBUF = 2
_NA = 10240
_NPAD = _NA - _N
_RPT = _NA // _NS


def _sc_conv_body(y_hbm, ridx_hbm, didx_hbm, out_hbm,
                  ridx_v, didx_v, rows, zbuf, acc,
                  g0, g1, s0, s1):
    gs = [g0, g1]
    ss = [s0, s1]
    c = lax.axis_index("c")
    s = lax.axis_index("s")
    wid = s * _NC + c

    base = wid * _CH
    pltpu.sync_copy(ridx_hbm.at[pl.ds(base, _CH)], ridx_v)
    pltpu.sync_copy(didx_hbm.at[pl.ds(base, _CH)], didx_v)
    pltpu.async_copy(y_hbm.at[ridx_v.at[0]], rows.at[0], gs[0])

    def zero_row(r, carry):
        for gg in range(_C // 16):
            zbuf[r, pl.ds(16 * gg, 16)] = jnp.zeros((16,), jnp.float32)
        return carry

    nz = zbuf.shape[0]
    lax.fori_loop(0, nz, zero_row, 0)
    for i in range(_RPT // nz):
        pltpu.sync_copy(zbuf, acc.at[pl.ds(s * _RPT + i * nz, nz)])

    plsc.subcore_barrier()

    n_grp = _CH // _NBUF

    def chunk_group(g, carry):
        for b in range(_NBUF):
            ob = 1 - b
            j = 2 * g + b

            @pl.when((g > 0) | (b == 1))
            def _drain():
                pltpu.make_async_copy(rows.at[ob], acc.at[didx_v.at[j - 1]],
                                      ss[ob]).wait()

            @pl.when((g < n_grp - 1) | (b == 0))
            def _refill():
                pltpu.async_copy(y_hbm.at[ridx_v.at[j + 1]], rows.at[ob],
                                 gs[ob])

            pltpu.make_async_copy(y_hbm.at[ridx_v.at[j]], rows.at[b],
                                  gs[b]).wait()
            pltpu.async_copy(rows.at[b], acc.at[didx_v.at[j]],
                             ss[b], add=True)
        return carry

    lax.fori_loop(0, n_grp, chunk_group, 0)
    pltpu.make_async_copy(rows.at[1], acc.at[didx_v.at[_CH - 1]],
                          ss[1]).wait()

    plsc.subcore_barrier()
    pltpu.sync_copy(acc.at[pl.ds(s * _RPT, _RPT)],
                    out_hbm.at[c, pl.ds(s * _RPT, _RPT)])


_sc_conv = pl.kernel(
    _sc_conv_body,
    out_type=jax.ShapeDtypeStruct((_NC, _NA, _C), jnp.float32),
    mesh=plsc.VectorSubcoreMesh(core_axis_name="c", subcore_axis_name="s"),
    scratch_types=[
        pltpu.VMEM((_CH, _G), jnp.int32),
        pltpu.VMEM((_CH, _G), jnp.int32),
        pltpu.VMEM((_NBUF, _G, _C), jnp.float32),
        pltpu.VMEM((32, _C), jnp.float32),
        pltpu.VMEM_SHARED((_NA, _C), jnp.float32),
    ] + [pltpu.SemaphoreType.DMA] * 4,
)


_KB = 3
_WB = _KB * _C


def _mm_body(x_ref, w_ref, y_ref, xb_ref):
    @pl.when(pl.program_id(0) == 0)
    def _cast():
        xb_ref[...] = x_ref[...].astype(jnp.bfloat16)

    y_ref[...] = jnp.dot(xb_ref[...], w_ref[...].astype(jnp.bfloat16),
                         preferred_element_type=jnp.float32)


def _transform(x, Wf):
    y = pl.pallas_call(
        _mm_body,
        grid=(_K // _KB,),
        in_specs=[pl.BlockSpec((_N, _C), lambda k: (0, 0)),
                  pl.BlockSpec((_C, _WB), lambda k: (0, k))],
        out_specs=pl.BlockSpec((_N, _WB), lambda k: (0, k)),
        out_shape=jax.ShapeDtypeStruct((_N, _K * _C), jnp.float32),
        scratch_shapes=[pltpu.VMEM((_N, _C), jnp.bfloat16)],
    )(x, Wf)
    return y.reshape(_K * _N, _C)


def _bn_mm_body(acc_ref, g_ref, b_ref, w_ref, y_ref, h_ref):
    @pl.when(pl.program_id(0) == 0)
    def _bn():
        h = acc_ref[0, :_N] + acc_ref[1, :_N]
        mu = jnp.mean(h, axis=0, keepdims=True)
        var = jnp.mean(jnp.square(h - mu), axis=0, keepdims=True)
        h_ref[...] = jnp.maximum(
            (h - mu) * lax.rsqrt(var + 1e-4) * g_ref[...] + b_ref[...],
            0.0).astype(jnp.bfloat16)

    y_ref[...] = jnp.dot(h_ref[...], w_ref[...].astype(jnp.bfloat16),
                         preferred_element_type=jnp.float32)


def _bn_transform(acc, g, b, Wf):
    y = pl.pallas_call(
        _bn_mm_body,
        grid=(_K // _KB,),
        in_specs=[pl.BlockSpec((_NC, _NA, _C), lambda k: (0, 0, 0)),
                  pl.BlockSpec((1, _C), lambda k: (0, 0)),
                  pl.BlockSpec((1, _C), lambda k: (0, 0)),
                  pl.BlockSpec((_C, _WB), lambda k: (0, k))],
        out_specs=pl.BlockSpec((_N, _WB), lambda k: (0, k)),
        out_shape=jax.ShapeDtypeStruct((_N, _K * _C), jnp.float32),
        scratch_shapes=[pltpu.VMEM((_N, _C), jnp.bfloat16)],
    )(acc, g.reshape(1, _C), b.reshape(1, _C), Wf)
    return y.reshape(_K * _N, _C)


def _bn_res_relu_body(acc_ref, g_ref, b_ref, x_ref, o_ref):
    h = acc_ref[0, :_N] + acc_ref[1, :_N]
    mu = jnp.mean(h, axis=0, keepdims=True)
    var = jnp.mean(jnp.square(h - mu), axis=0, keepdims=True)
    o_ref[...] = jnp.maximum(
        (h - mu) * lax.rsqrt(var + 1e-4) * g_ref[...] + b_ref[...]
        + x_ref[...], 0.0)


def _bn_res_relu(acc, g, b, x):
    return pl.pallas_call(
        _bn_res_relu_body,
        out_shape=jax.ShapeDtypeStruct((_N, _C), jnp.float32),
    )(acc, g.reshape(1, _C), b.reshape(1, _C), x)


def kernel(x, W1, g1, b1, W2, g2, b2, edge_index, kernel_idx):
    src = edge_index[0].astype(jnp.int32)
    dst = edge_index[1].astype(jnp.int32)
    kidx = kernel_idx.astype(jnp.int32)

    row_id = jnp.arange(_ROWS, dtype=jnp.int32)[:, None]
    gpad = jnp.broadcast_to(row_id, (_ROWS, _G - _GR))
    dpad = _N + row_id % _NPAD
    dpad = jnp.broadcast_to(dpad, (_ROWS, _G - _GR))
    ridx = jnp.concatenate(
        [(src * _K + kidx).reshape(_ROWS, _GR), gpad], axis=1)
    didx = jnp.concatenate([dst.reshape(_ROWS, _GR), dpad], axis=1)

    wf1 = W1.transpose(1, 0, 2).reshape(_C, _K * _C)
    wf2 = W2.transpose(1, 0, 2).reshape(_C, _K * _C)
    y1 = _transform(x, wf1)
    acc1 = _sc_conv(y1, ridx, didx)
    y2 = _bn_transform(acc1, g1, b1, wf2)
    acc2 = _sc_conv(y2, ridx, didx)
    return _bn_res_relu(acc2, g2, b2, x)


# speedup vs baseline: 2.0081x; 2.0081x over previous
import jax
import jax.numpy as jnp
from jax import lax
from jax.experimental import pallas as pl
from jax.experimental.pallas import tpu as pltpu
from jax.experimental.pallas import tpu_sc as plsc

_N = 10000
_E = 160000
_C = 128
_K = 27

_NC = 2
_NS = 16
_NW = _NC * _NS
_G = 128
_GR = 125
_ROWS = _E // _GR
_CH = _ROWS // _NW
_NBUF = 2
_NA = 10240
_NPAD = _NA - _N
_RPT = _NA // _NS


def _sc_conv_body(y_hbm, ridx_hbm, didx_hbm, out_hbm,
                  ridx_v, didx_v, rows, zbuf, acc,
                  g0, g1, s0, s1):
    gs = [g0, g1]
    ss = [s0, s1]
    c = lax.axis_index("c")
    s = lax.axis_index("s")
    wid = s * _NC + c

    base = wid * _CH
    pltpu.sync_copy(ridx_hbm.at[pl.ds(base, _CH)], ridx_v)
    pltpu.sync_copy(didx_hbm.at[pl.ds(base, _CH)], didx_v)
    pltpu.async_copy(y_hbm.at[ridx_v.at[0]], rows.at[0], gs[0])

    def zero_row(r, carry):
        for gg in range(_C // 16):
            zbuf[r, pl.ds(16 * gg, 16)] = jnp.zeros((16,), jnp.float32)
        return carry

    nz = zbuf.shape[0]
    lax.fori_loop(0, nz, zero_row, 0)
    for i in range(_RPT // nz):
        pltpu.sync_copy(zbuf, acc.at[pl.ds(s * _RPT + i * nz, nz)])

    plsc.subcore_barrier()

    n_grp = _CH // _NBUF

    def chunk_group(g, carry):
        for b in range(_NBUF):
            ob = 1 - b
            j = 2 * g + b

            @pl.when((g > 0) | (b == 1))
            def _drain():
                pltpu.make_async_copy(rows.at[ob], acc.at[didx_v.at[j - 1]],
                                      ss[ob]).wait()

            @pl.when((g < n_grp - 1) | (b == 0))
            def _refill():
                pltpu.async_copy(y_hbm.at[ridx_v.at[j + 1]], rows.at[ob],
                                 gs[ob])

            pltpu.make_async_copy(y_hbm.at[ridx_v.at[j]], rows.at[b],
                                  gs[b]).wait()
            pltpu.async_copy(rows.at[b], acc.at[didx_v.at[j]],
                             ss[b], add=True)
        return carry

    lax.fori_loop(0, n_grp, chunk_group, 0)
    pltpu.make_async_copy(rows.at[1], acc.at[didx_v.at[_CH - 1]],
                          ss[1]).wait()

    plsc.subcore_barrier()
    pltpu.sync_copy(acc.at[pl.ds(s * _RPT, _RPT)],
                    out_hbm.at[c, pl.ds(s * _RPT, _RPT)])


_sc_conv = pl.kernel(
    _sc_conv_body,
    out_type=jax.ShapeDtypeStruct((_NC, _NA, _C), jnp.float32),
    mesh=plsc.VectorSubcoreMesh(core_axis_name="c", subcore_axis_name="s"),
    scratch_types=[
        pltpu.VMEM((_CH, _G), jnp.int32),
        pltpu.VMEM((_CH, _G), jnp.int32),
        pltpu.VMEM((_NBUF, _G, _C), jnp.float32),
        pltpu.VMEM((32, _C), jnp.float32),
        pltpu.VMEM_SHARED((_NA, _C), jnp.float32),
    ] + [pltpu.SemaphoreType.DMA] * 4,
)


def _mm_body(x_ref, w_ref, y_ref):
    y_ref[0] = jnp.dot(x_ref[...], w_ref[0],
                       preferred_element_type=jnp.float32)


def _transform(x, W):
    y = pl.pallas_call(
        _mm_body,
        grid=(_K,),
        in_specs=[pl.BlockSpec((_N, _C), lambda k: (0, 0)),
                  pl.BlockSpec((1, _C, _C), lambda k: (k, 0, 0))],
        out_specs=pl.BlockSpec((1, _N, _C), lambda k: (k, 0, 0)),
        out_shape=jax.ShapeDtypeStruct((_K, _N, _C), jnp.float32),
    )(x, W)
    return y.reshape(_K * _N, _C)


def _bn_mm_body(acc_ref, g_ref, b_ref, w_ref, y_ref, h_ref):
    @pl.when(pl.program_id(0) == 0)
    def _bn():
        h = acc_ref[0, :_N] + acc_ref[1, :_N]
        mu = jnp.mean(h, axis=0, keepdims=True)
        var = jnp.mean(jnp.square(h - mu), axis=0, keepdims=True)
        h_ref[...] = jnp.maximum(
            (h - mu) * lax.rsqrt(var + 1e-4) * g_ref[...] + b_ref[...], 0.0)

    y_ref[0] = jnp.dot(h_ref[...], w_ref[0],
                       preferred_element_type=jnp.float32)


def _bn_transform(acc, g, b, W):
    y = pl.pallas_call(
        _bn_mm_body,
        grid=(_K,),
        in_specs=[pl.BlockSpec((_NC, _NA, _C), lambda k: (0, 0, 0)),
                  pl.BlockSpec((1, _C), lambda k: (0, 0)),
                  pl.BlockSpec((1, _C), lambda k: (0, 0)),
                  pl.BlockSpec((1, _C, _C), lambda k: (k, 0, 0))],
        out_specs=pl.BlockSpec((1, _N, _C), lambda k: (k, 0, 0)),
        out_shape=jax.ShapeDtypeStruct((_K, _N, _C), jnp.float32),
        scratch_shapes=[pltpu.VMEM((_N, _C), jnp.float32)],
    )(acc, g.reshape(1, _C), b.reshape(1, _C), W)
    return y.reshape(_K * _N, _C)


def _bn_res_relu_body(acc_ref, g_ref, b_ref, x_ref, o_ref):
    h = acc_ref[0, :_N] + acc_ref[1, :_N]
    mu = jnp.mean(h, axis=0, keepdims=True)
    var = jnp.mean(jnp.square(h - mu), axis=0, keepdims=True)
    o_ref[...] = jnp.maximum(
        (h - mu) * lax.rsqrt(var + 1e-4) * g_ref[...] + b_ref[...]
        + x_ref[...], 0.0)


def _bn_res_relu(acc, g, b, x):
    return pl.pallas_call(
        _bn_res_relu_body,
        out_shape=jax.ShapeDtypeStruct((_N, _C), jnp.float32),
    )(acc, g.reshape(1, _C), b.reshape(1, _C), x)


def kernel(x, W1, g1, b1, W2, g2, b2, edge_index, kernel_idx):
    src = edge_index[0].astype(jnp.int32)
    dst = edge_index[1].astype(jnp.int32)
    kidx = kernel_idx.astype(jnp.int32)

    row_id = jnp.arange(_ROWS, dtype=jnp.int32)[:, None]
    gpad = jnp.broadcast_to(row_id, (_ROWS, _G - _GR))
    dpad = _N + row_id % _NPAD
    dpad = jnp.broadcast_to(dpad, (_ROWS, _G - _GR))
    ridx = jnp.concatenate(
        [(kidx * _N + src).reshape(_ROWS, _GR), gpad], axis=1)
    didx = jnp.concatenate([dst.reshape(_ROWS, _GR), dpad], axis=1)

    y1 = _transform(x, W1)
    acc1 = _sc_conv(y1, ridx, didx)
    y2 = _bn_transform(acc1, g1, b1, W2)
    acc2 = _sc_conv(y2, ridx, didx)
    return _bn_res_relu(acc2, g2, b2, x)


# speedup vs baseline: 2.0215x; 1.0067x over previous
import jax
import jax.numpy as jnp
from jax import lax
from jax.experimental import pallas as pl
from jax.experimental.pallas import tpu as pltpu
from jax.experimental.pallas import tpu_sc as plsc

_N = 10000
_E = 160000
_C = 128
_K = 27

_NC = 2
_NS = 16
_NW = _NC * _NS
_G = 128
_GR = 125
_ROWS = _E // _GR
_CH = _ROWS // _NW
_NBUF = 2
_NA = 10240
_NPAD = _NA - _N
_RPT = _NA // _NS


def _sc_conv_body(y_hbm, ridx_hbm, didx_hbm, out_hbm,
                  ridx_v, didx_v, rows, zbuf, acc,
                  g0, g1, s0, s1):
    gs = [g0, g1]
    ss = [s0, s1]
    c = lax.axis_index("c")
    s = lax.axis_index("s")
    wid = s * _NC + c

    base = wid * _CH
    pltpu.sync_copy(ridx_hbm.at[pl.ds(base, _CH)], ridx_v)
    pltpu.sync_copy(didx_hbm.at[pl.ds(base, _CH)], didx_v)
    for b in range(_NBUF):
        pltpu.async_copy(y_hbm.at[ridx_v.at[b]], rows.at[b], gs[b])

    def zero_row(r, carry):
        for gg in range(_C // 16):
            zbuf[r, pl.ds(16 * gg, 16)] = jnp.zeros((16,), jnp.float32)
        return carry

    nz = zbuf.shape[0]
    lax.fori_loop(0, nz, zero_row, 0)
    for i in range(_RPT // nz):
        pltpu.sync_copy(zbuf, acc.at[pl.ds(s * _RPT + i * nz, nz)])

    plsc.subcore_barrier()

    n_grp = _CH // _NBUF

    def chunk_group(g, carry):
        for b in range(_NBUF):
            ob = 1 - b
            j = 2 * g + b

            @pl.when((g > 0) | (b == 1))
            def _drain():
                pltpu.make_async_copy(rows.at[ob], acc.at[didx_v.at[j - 1]],
                                      ss[ob]).wait()

            @pl.when(((g > 0) | (b == 1)) & ((g < n_grp - 1) | (b == 0)))
            def _refill():
                pltpu.async_copy(y_hbm.at[ridx_v.at[j + 1]], rows.at[ob],
                                 gs[ob])

            pltpu.make_async_copy(y_hbm.at[ridx_v.at[j]], rows.at[b],
                                  gs[b]).wait()
            pltpu.async_copy(rows.at[b], acc.at[didx_v.at[j]],
                             ss[b], add=True)
        return carry

    lax.fori_loop(0, n_grp, chunk_group, 0)
    pltpu.make_async_copy(rows.at[1], acc.at[didx_v.at[_CH - 1]],
                          ss[1]).wait()

    plsc.subcore_barrier()
    pltpu.sync_copy(acc.at[pl.ds(s * _RPT, _RPT)],
                    out_hbm.at[c, pl.ds(s * _RPT, _RPT)])


_sc_conv = pl.kernel(
    _sc_conv_body,
    out_type=jax.ShapeDtypeStruct((_NC, _NA, _C), jnp.float32),
    mesh=plsc.VectorSubcoreMesh(core_axis_name="c", subcore_axis_name="s"),
    scratch_types=[
        pltpu.VMEM((_CH, _G), jnp.int32),
        pltpu.VMEM((_CH, _G), jnp.int32),
        pltpu.VMEM((_NBUF, _G, _C), jnp.float32),
        pltpu.VMEM((32, _C), jnp.float32),
        pltpu.VMEM_SHARED((_NA, _C), jnp.float32),
    ] + [pltpu.SemaphoreType.DMA] * 4,
)


def _mm_body(x_ref, w_ref, y_ref):
    y_ref[0] = jnp.dot(x_ref[...], w_ref[0],
                       preferred_element_type=jnp.float32)


def _transform(x, W):
    y = pl.pallas_call(
        _mm_body,
        grid=(_K,),
        in_specs=[pl.BlockSpec((_N, _C), lambda k: (0, 0)),
                  pl.BlockSpec((1, _C, _C), lambda k: (k, 0, 0))],
        out_specs=pl.BlockSpec((1, _N, _C), lambda k: (k, 0, 0)),
        out_shape=jax.ShapeDtypeStruct((_K, _N, _C), jnp.float32),
    )(x, W)
    return y.reshape(_K * _N, _C)


def _bn_mm_body(acc_ref, g_ref, b_ref, w_ref, y_ref, h_ref):
    @pl.when(pl.program_id(0) == 0)
    def _bn():
        h = acc_ref[0, :_N] + acc_ref[1, :_N]
        mu = jnp.mean(h, axis=0, keepdims=True)
        var = jnp.mean(jnp.square(h - mu), axis=0, keepdims=True)
        h_ref[...] = jnp.maximum(
            (h - mu) * lax.rsqrt(var + 1e-4) * g_ref[...] + b_ref[...], 0.0)

    y_ref[0] = jnp.dot(h_ref[...], w_ref[0],
                       preferred_element_type=jnp.float32)


def _bn_transform(acc, g, b, W):
    y = pl.pallas_call(
        _bn_mm_body,
        grid=(_K,),
        in_specs=[pl.BlockSpec((_NC, _NA, _C), lambda k: (0, 0, 0)),
                  pl.BlockSpec((1, _C), lambda k: (0, 0)),
                  pl.BlockSpec((1, _C), lambda k: (0, 0)),
                  pl.BlockSpec((1, _C, _C), lambda k: (k, 0, 0))],
        out_specs=pl.BlockSpec((1, _N, _C), lambda k: (k, 0, 0)),
        out_shape=jax.ShapeDtypeStruct((_K, _N, _C), jnp.float32),
        scratch_shapes=[pltpu.VMEM((_N, _C), jnp.float32)],
    )(acc, g.reshape(1, _C), b.reshape(1, _C), W)
    return y.reshape(_K * _N, _C)


def _bn_res_relu_body(acc_ref, g_ref, b_ref, x_ref, o_ref):
    h = acc_ref[0, :_N] + acc_ref[1, :_N]
    mu = jnp.mean(h, axis=0, keepdims=True)
    var = jnp.mean(jnp.square(h - mu), axis=0, keepdims=True)
    o_ref[...] = jnp.maximum(
        (h - mu) * lax.rsqrt(var + 1e-4) * g_ref[...] + b_ref[...]
        + x_ref[...], 0.0)


def _bn_res_relu(acc, g, b, x):
    return pl.pallas_call(
        _bn_res_relu_body,
        out_shape=jax.ShapeDtypeStruct((_N, _C), jnp.float32),
    )(acc, g.reshape(1, _C), b.reshape(1, _C), x)


def kernel(x, W1, g1, b1, W2, g2, b2, edge_index, kernel_idx):
    src = edge_index[0].astype(jnp.int32)
    dst = edge_index[1].astype(jnp.int32)
    kidx = kernel_idx.astype(jnp.int32)

    row_id = jnp.arange(_ROWS, dtype=jnp.int32)[:, None]
    gpad = jnp.broadcast_to(row_id, (_ROWS, _G - _GR))
    dpad = _N + row_id % _NPAD
    dpad = jnp.broadcast_to(dpad, (_ROWS, _G - _GR))
    ridx = jnp.concatenate(
        [(kidx * _N + src).reshape(_ROWS, _GR), gpad], axis=1)
    didx = jnp.concatenate([dst.reshape(_ROWS, _GR), dpad], axis=1)

    y1 = _transform(x, W1)
    acc1 = _sc_conv(y1, ridx, didx)
    y2 = _bn_transform(acc1, g1, b1, W2)
    acc2 = _sc_conv(y2, ridx, didx)
    return _bn_res_relu(acc2, g2, b2, x)
